# Initial kernel scaffold; baseline (speedup 1.0000x reference)
import jax
import jax.numpy as jnp
from jax.experimental import pallas as pl


def kernel(x, edge_index, batch, W1, b1, W2, b2, Wfc, bfc, Wh, bh):
    raise NotImplementedError("write your pallas kernel here")


# speedup vs baseline: 12.0390x; 12.0390x over previous
import functools

import jax
import jax.numpy as jnp
from jax import lax
from jax.experimental import pallas as pl
from jax.experimental.pallas import tpu as pltpu
from jax.experimental.pallas import tpu_sc as plsc

N = 10000
E = 320000
D = 128
H = 128
T = 8
G = 128

NC = 2
NS = 16
NW = NC * NS
CHUNK = 128
EPT = E // NW
NCHUNK = -(-EPT // CHUNK)
EPT_PAD = NCHUNK * CHUNK
E_PAD = EPT_PAD * NW
ROWS_PER_TILE = 632
ACC_ROWS = ROWS_PER_TILE * NS
DEG_W = 128
BATW = 16
BR = 1000


def _sc_degree(dst3, zeros_deg, ones_deg):
    mesh = plsc.VectorSubcoreMesh(core_axis_name="c", subcore_axis_name="s")

    @functools.partial(
        pl.kernel,
        mesh=mesh,
        out_type=jax.ShapeDtypeStruct((NC, ACC_ROWS, DEG_W), jnp.float32),
        scratch_types=[
            pltpu.VMEM((NCHUNK, CHUNK), jnp.int32),
            pltpu.VMEM((CHUNK, DEG_W), jnp.float32),
            pltpu.VMEM_SHARED((ACC_ROWS, DEG_W), jnp.float32),
            pltpu.SemaphoreType.DMA,
        ],
    )
    def k(dst_hbm, z_hbm, o_hbm, out_hbm, idx_v, ones_v, acc, sem):
        cid = lax.axis_index("c")
        sid = lax.axis_index("s")
        wid = sid * NC + cid
        my_rows = pl.ds(sid * ROWS_PER_TILE, ROWS_PER_TILE)
        pltpu.sync_copy(z_hbm, acc.at[my_rows])
        pltpu.sync_copy(dst_hbm.at[wid], idx_v)
        pltpu.sync_copy(o_hbm, ones_v)
        plsc.subcore_barrier()

        def body(j, carry):
            pltpu.sync_copy(ones_v, acc.at[idx_v.at[j]], add=True)
            return carry

        lax.fori_loop(0, NCHUNK, body, 0)
        plsc.subcore_barrier()
        pltpu.sync_copy(acc.at[my_rows], out_hbm.at[cid, my_rows])

    return k(dst3, zeros_deg, ones_deg)


def _sc_scatter(hs, src3, dst3, zeros_rows):
    mesh = plsc.VectorSubcoreMesh(core_axis_name="c", subcore_axis_name="s")

    @functools.partial(
        pl.kernel,
        mesh=mesh,
        out_type=jax.ShapeDtypeStruct((NC, ACC_ROWS, H), jnp.float32),
        scratch_types=[
            pltpu.VMEM((NCHUNK, CHUNK), jnp.int32),
            pltpu.VMEM((NCHUNK, CHUNK), jnp.int32),
            pltpu.VMEM((CHUNK, H), jnp.float32),
            pltpu.VMEM_SHARED((ACC_ROWS, H), jnp.float32),
            pltpu.SemaphoreType.DMA,
        ],
    )
    def k(hs_hbm, src_hbm, dst_hbm, z_hbm, out_hbm, src_v, dst_v, rows_v, acc, sem):
        cid = lax.axis_index("c")
        sid = lax.axis_index("s")
        wid = sid * NC + cid
        my_rows = pl.ds(sid * ROWS_PER_TILE, ROWS_PER_TILE)
        pltpu.sync_copy(z_hbm, acc.at[my_rows])
        pltpu.sync_copy(src_hbm.at[wid], src_v)
        pltpu.sync_copy(dst_hbm.at[wid], dst_v)
        plsc.subcore_barrier()

        def body(j, carry):
            pltpu.async_copy(hs_hbm.at[src_v.at[j]], rows_v, sem).wait()
            pltpu.sync_copy(rows_v, acc.at[dst_v.at[j]], add=True)
            return carry

        lax.fori_loop(0, NCHUNK, body, 0)
        plsc.subcore_barrier()
        pltpu.sync_copy(acc.at[my_rows], out_hbm.at[cid, my_rows])

    return k(hs, src3, dst3, zeros_rows)


def _tc_first(x, p0, p1, W1):
    def body(x_ref, p0_ref, p1_ref, w_ref, hs_ref, dis_ref):
        deg = p0_ref[:, 0:1] + p1_ref[:, 0:1] + 1.0
        dis = lax.rsqrt(deg)
        h = jnp.dot(x_ref[...], w_ref[...], preferred_element_type=jnp.float32)
        hs_ref[...] = h * dis
        dis_ref[...] = jnp.broadcast_to(dis, dis_ref.shape)

    return pl.pallas_call(
        body,
        grid=(N // BR,),
        in_specs=[
            pl.BlockSpec((BR, D), lambda i: (i, 0)),
            pl.BlockSpec((BR, DEG_W), lambda i: (i, 0)),
            pl.BlockSpec((BR, DEG_W), lambda i: (i, 0)),
            pl.BlockSpec((D, H), lambda i: (0, 0)),
        ],
        out_specs=[
            pl.BlockSpec((BR, H), lambda i: (i, 0)),
            pl.BlockSpec((BR, H), lambda i: (i, 0)),
        ],
        out_shape=[
            jax.ShapeDtypeStruct((N, H), jnp.float32),
            jax.ShapeDtypeStruct((N, H), jnp.float32),
        ],
    )(x, p0, p1, W1)


def _tc_mid(q0, q1, hs1, dis2d, b1r, W2):
    def body(q0_ref, q1_ref, hs_ref, dis_ref, b_ref, w_ref, out_ref):
        t = q0_ref[...] + q1_ref[...] + hs_ref[...]
        t = jnp.maximum(dis_ref[...] * t + b_ref[...], 0.0)
        h2 = jnp.dot(t, w_ref[...], preferred_element_type=jnp.float32)
        out_ref[...] = h2 * dis_ref[...]

    return pl.pallas_call(
        body,
        grid=(N // BR,),
        in_specs=[
            pl.BlockSpec((BR, H), lambda i: (i, 0)),
            pl.BlockSpec((BR, H), lambda i: (i, 0)),
            pl.BlockSpec((BR, H), lambda i: (i, 0)),
            pl.BlockSpec((BR, H), lambda i: (i, 0)),
            pl.BlockSpec((1, H), lambda i: (0, 0)),
            pl.BlockSpec((H, H), lambda i: (0, 0)),
        ],
        out_specs=pl.BlockSpec((BR, H), lambda i: (i, 0)),
        out_shape=jax.ShapeDtypeStruct((N, H), jnp.float32),
    )(q0, q1, hs1, dis2d, b1r, W2)


def _tc_final(q0, q1, hs2, dis2d, b2r, batchw, Wfc, bfcr, Whr, bhr):
    nsteps = N // BR

    def body(q0_ref, q1_ref, hs_ref, dis_ref, b_ref, batch_ref,
             wfc_ref, bfc_ref, wh_ref, bh_ref, out_ref, psum, cnt):
        step = pl.program_id(0)

        @pl.when(step == 0)
        def _():
            psum[...] = jnp.zeros_like(psum)
            cnt[...] = jnp.zeros_like(cnt)

        o = q0_ref[...] + q1_ref[...] + hs_ref[...]
        o = jnp.maximum(dis_ref[...] * o + b_ref[...], 0.0)
        bidx = batch_ref[:, 0:1]
        gids = lax.broadcasted_iota(jnp.int32, (1, G), 1)
        onehot = (bidx == gids).astype(jnp.float32)
        psum[...] += lax.dot_general(
            onehot, o, (((0,), (0,)), ((), ())),
            preferred_element_type=jnp.float32)
        cnt[...] += lax.dot_general(
            onehot, jnp.ones((BR, H), jnp.float32), (((0,), (0,)), ((), ())),
            preferred_element_type=jnp.float32)

        @pl.when(step == nsteps - 1)
        def _():
            pooled = psum[...] / jnp.maximum(cnt[...], 1.0)
            z = jnp.dot(pooled, wfc_ref[...], preferred_element_type=jnp.float32)
            z = jnp.maximum(z + bfc_ref[...], 0.0)
            out_ref[...] = jnp.dot(
                z, wh_ref[...], preferred_element_type=jnp.float32) + bh_ref[...]

    return pl.pallas_call(
        body,
        grid=(nsteps,),
        in_specs=[
            pl.BlockSpec((BR, H), lambda i: (i, 0)),
            pl.BlockSpec((BR, H), lambda i: (i, 0)),
            pl.BlockSpec((BR, H), lambda i: (i, 0)),
            pl.BlockSpec((BR, H), lambda i: (i, 0)),
            pl.BlockSpec((1, H), lambda i: (0, 0)),
            pl.BlockSpec((BR, BATW), lambda i: (i, 0)),
            pl.BlockSpec((H, H), lambda i: (0, 0)),
            pl.BlockSpec((1, H), lambda i: (0, 0)),
            pl.BlockSpec((H, T * 2), lambda i: (0, 0)),
            pl.BlockSpec((1, T * 2), lambda i: (0, 0)),
        ],
        out_specs=pl.BlockSpec((G, T * 2), lambda i: (0, 0)),
        out_shape=jax.ShapeDtypeStruct((G, T * 2), jnp.float32),
        scratch_shapes=[
            pltpu.VMEM((G, H), jnp.float32),
            pltpu.VMEM((G, H), jnp.float32),
        ],
    )(q0, q1, hs2, dis2d, b2r, batchw, Wfc, bfcr, Whr, bhr)


def kernel(x, edge_index, batch, W1, b1, W2, b2, Wfc, bfc, Wh, bh):
    pad = E_PAD - E
    src3 = jnp.concatenate(
        [edge_index[0], jnp.zeros((pad,), jnp.int32)]).reshape(NW, NCHUNK, CHUNK)
    dst3 = jnp.concatenate(
        [edge_index[1], jnp.full((pad,), N, jnp.int32)]).reshape(NW, NCHUNK, CHUNK)
    zeros_deg = jnp.zeros((ROWS_PER_TILE, DEG_W), jnp.float32)
    ones_deg = jnp.ones((CHUNK, DEG_W), jnp.float32)
    zeros_rows = jnp.zeros((ROWS_PER_TILE, H), jnp.float32)

    degp = _sc_degree(dst3, zeros_deg, ones_deg)
    hs1, dis2d = _tc_first(x, degp[0, :N, :], degp[1, :N, :], W1)
    m1 = _sc_scatter(hs1, src3, dst3, zeros_rows)
    hs2 = _tc_mid(m1[0, :N, :], m1[1, :N, :], hs1, dis2d,
                  b1.reshape(1, H), W2)
    m2 = _sc_scatter(hs2, src3, dst3, zeros_rows)
    batchw = jnp.broadcast_to(batch[:, None], (N, BATW))
    Whr = Wh.transpose(1, 0, 2).reshape(H, T * 2)
    outf = _tc_final(m2[0, :N, :], m2[1, :N, :], hs2, dis2d,
                     b2.reshape(1, H), batchw, Wfc, bfc.reshape(1, H),
                     Whr, bh.reshape(1, T * 2))
    return outf.reshape(G, T, 2).transpose(1, 0, 2)
